# Initial kernel scaffold; baseline (speedup 1.0000x reference)
import jax
import jax.numpy as jnp
from jax.experimental import pallas as pl


def kernel(inputs, A, edge_index, node_idx, g_ei1, g_ei2, params):
    raise NotImplementedError("write your pallas kernel here")


# speedup vs baseline: 7.7824x; 7.7824x over previous
import functools

import jax
import jax.numpy as jnp
from jax import lax
from jax.experimental import pallas as pl
from jax.experimental.pallas import tpu as pltpu
from jax.experimental.pallas import tpu_sc as plsc

H = 128
NC, NS = 2, 16
NW = NC * NS
C = 64
CG = 128

NSESS = 10000
NPAD_S = 10240
NG = 4096
NPAD_G = 4608

BGATHER = 16384

E_SESS = 320000
E_GAT_S = E_SESS + NSESS
EPAD_GAT_S = 162 * NW * C
EPAD_GGC = 157 * NW * C
E_G2 = 100000
E_GAT_G = E_G2 + NG
EPAD_GAT_G = 51 * NW * C
EPAD_SAGE = 49 * NW * C


def _mesh():
    return plsc.VectorSubcoreMesh(
        core_axis_name="c", subcore_axis_name="s", num_cores=NC, num_subcores=NS)


def _make_gather(B):
    bpw = B // NW
    nch = bpw // CG

    @functools.partial(
        pl.kernel,
        out_type=jax.ShapeDtypeStruct((B, H), jnp.float32),
        mesh=_mesh(),
        compiler_params=pltpu.CompilerParams(needs_layout_passes=False),
        scratch_types=[
            pltpu.VMEM((nch, CG), jnp.int32),
            pltpu.VMEM((bpw, H), jnp.float32),
            pltpu.SemaphoreType.DMA,
        ],
    )
    def gather_k(table, idx_hbm, out, idx_v, rows_v, sem):
        wid = lax.axis_index("s") * NC + lax.axis_index("c")
        base = wid * bpw
        for j in range(nch):
            pltpu.sync_copy(idx_hbm.at[pl.ds(base + j * CG, CG)], idx_v.at[j])
        cps = [
            pltpu.async_copy(table.at[idx_v.at[j]], rows_v.at[pl.ds(j * CG, CG)], sem)
            for j in range(nch)
        ]
        for cp in cps:
            cp.wait()
        pltpu.sync_copy(rows_v, out.at[pl.ds(base, bpw)])

    return gather_k


def _make_agg(npad, epad, natt, att):
    kchunks = epad // (NW * C)
    stripe = npad // NS

    scratch = [
        pltpu.VMEM((C,), jnp.int32),
        pltpu.VMEM((C,), jnp.int32),
        pltpu.VMEM((C, H), jnp.float32),
        pltpu.VMEM((C, 16), jnp.float32),
        pltpu.VMEM((16,), jnp.float32),
    ]
    if att:
        scratch += [
            pltpu.VMEM((natt,), jnp.float32),
            pltpu.VMEM((natt,), jnp.float32),
        ]
    scratch += [
        pltpu.VMEM_SHARED((npad, H), jnp.float32),
        pltpu.VMEM_SHARED((npad, 16), jnp.float32),
        pltpu.SemaphoreType.DMA,
    ]

    def body(refs):
        if att:
            (xw_hbm, asrc_hbm, adst_hbm, src_hbm, dst_hbm,
             num_out, den_out,
             src_v, dst_v, rows_v, den_v, wtmp_v, asrc_v, adst_v,
             sh_num, sh_den, sem) = refs
        else:
            (xw_hbm, src_hbm, dst_hbm,
             num_out, den_out,
             src_v, dst_v, rows_v, den_v, wtmp_v,
             sh_num, sh_den, sem) = refs

        cid = lax.axis_index("c")
        sid = lax.axis_index("s")
        wid = sid * NC + cid
        r0 = sid * stripe
        CB = 32
        nb = stripe // CB
        zero16f = jnp.zeros((16,), jnp.float32)

        if att:
            pltpu.sync_copy(asrc_hbm, asrc_v)
            pltpu.sync_copy(adst_hbm, adst_v)

        def zrow(r, carry):
            for cc in range(8):
                rows_v[r, pl.ds(cc * 16, 16)] = zero16f
            den_v[r, :] = zero16f
            return carry

        lax.fori_loop(0, C, zrow, 0)
        for t in range(nb):
            pltpu.sync_copy(rows_v.at[pl.ds(0, CB)],
                            sh_num.at[pl.ds(r0 + t * CB, CB)])
            pltpu.sync_copy(den_v.at[pl.ds(0, CB)],
                            sh_den.at[pl.ds(r0 + t * CB, CB)])

        lane = lax.iota(jnp.int32, 16)
        if not att:
            fill = jnp.where(lane == 0, jnp.float32(1.0), jnp.float32(0.0))

            def den_init(r, carry):
                den_v[r, :] = fill
                return carry

            lax.fori_loop(0, C, den_init, 0)
        plsc.subcore_barrier()

        def chunk_body(j, carry):
            off = (wid * kchunks + j) * C
            pltpu.sync_copy(src_hbm.at[pl.ds(off, C)], src_v)
            pltpu.sync_copy(dst_hbm.at[pl.ds(off, C)], dst_v)
            pltpu.async_copy(xw_hbm.at[src_v], rows_v, sem).wait()

            if att:
                zero16 = jnp.zeros((16,), jnp.int32)
                for g in range(C // 16):
                    si = src_v[pl.ds(g * 16, 16)]
                    di = dst_v[pl.ds(g * 16, 16)]
                    e = plsc.load_gather(asrc_v, [si]) + plsc.load_gather(adst_v, [di])
                    e = jnp.where(e >= 0.0, e, 0.2 * e)
                    w = jnp.exp(e)
                    plsc.store_scatter(den_v, [g * 16 + lane, zero16], w)
                    wtmp_v[...] = w

                    def edge_body(i, ecarry):
                        wb = plsc.load_gather(wtmp_v, [jnp.full((16,), 0, jnp.int32) + i])
                        r = g * 16 + i
                        for cc in range(8):
                            rows_v[r, pl.ds(cc * 16, 16)] = (
                                rows_v[r, pl.ds(cc * 16, 16)] * wb)
                        return ecarry

                    lax.fori_loop(0, 16, edge_body, 0)

            pltpu.sync_copy(rows_v, sh_num.at[dst_v], add=True)
            pltpu.sync_copy(den_v, sh_den.at[dst_v], add=True)
            return carry

        lax.fori_loop(0, kchunks, chunk_body, 0)
        plsc.subcore_barrier()

        for t in range(nb):
            rr = r0 + t * CB
            pltpu.sync_copy(sh_num.at[pl.ds(rr, CB)], rows_v.at[pl.ds(0, CB)])
            pltpu.sync_copy(rows_v.at[pl.ds(0, CB)],
                            num_out.at[cid, pl.ds(rr, CB)])
            pltpu.sync_copy(sh_den.at[pl.ds(rr, CB)], den_v.at[pl.ds(0, CB)])
            pltpu.sync_copy(den_v.at[pl.ds(0, CB)],
                            den_out.at[cid, pl.ds(rr, CB)])

    def kern(*refs):
        body(refs)

    return functools.partial(
        pl.kernel,
        out_type=[
            jax.ShapeDtypeStruct((NC, npad, H), jnp.float32),
            jax.ShapeDtypeStruct((NC, npad, 16), jnp.float32),
        ],
        mesh=_mesh(),
        compiler_params=pltpu.CompilerParams(needs_layout_passes=False,
                                             use_tc_tiling_on_sc=False),
        scratch_types=scratch,
    )(kern)


def _rows_spec(bs, width=H):
    return pl.BlockSpec((bs, width), lambda i: (i, 0))


def _full_spec(shape):
    return pl.BlockSpec(shape, lambda i: tuple(0 for _ in shape))


def _part_spec(bs, width):
    return pl.BlockSpec((NC, bs, width), lambda i: (0, i, 0))


def _gat_prep(x, w, asv, adv, npad, bs):
    grid = (npad // bs,)

    def body(x_ref, w_ref, asv_ref, adv_ref, xw_ref, as_ref, ad_ref):
        xw = jnp.dot(x_ref[...], w_ref[...], preferred_element_type=jnp.float32)
        xw_ref[...] = xw
        as_ref[...] = jnp.sum(xw * asv_ref[...], axis=1)
        ad_ref[...] = jnp.sum(xw * adv_ref[...], axis=1)

    return pl.pallas_call(
        body,
        grid=grid,
        in_specs=[_rows_spec(bs), _full_spec((H, H)), _full_spec((1, H)),
                  _full_spec((1, H))],
        out_specs=[_rows_spec(bs), pl.BlockSpec((bs,), lambda i: (i,)),
                   pl.BlockSpec((bs,), lambda i: (i,))],
        out_shape=[jax.ShapeDtypeStruct((npad, H), jnp.float32),
                   jax.ShapeDtypeStruct((npad,), jnp.float32),
                   jax.ShapeDtypeStruct((npad,), jnp.float32)],
    )(x, w, asv, adv)


def _sess_combine(num, den, b, w0, whhT, bhh, npad, bs):
    grid = (npad // bs,)

    def body(num_ref, den_ref, b_ref, w0_ref, whhT_ref, bhh_ref,
             h_ref, y1_ref, gh_ref):
        nm = num_ref[0] + num_ref[1]
        dn = den_ref[0][:, 0:1] + den_ref[1][:, 0:1]
        h = nm / (dn + 1e-16) + b_ref[...]
        h = jnp.maximum(h, 0.0)
        h_ref[...] = h
        y1_ref[...] = jnp.dot(h, w0_ref[...], preferred_element_type=jnp.float32)
        gh_ref[...] = (jnp.dot(h, whhT_ref[...], preferred_element_type=jnp.float32)
                       + bhh_ref[...])

    return pl.pallas_call(
        body,
        grid=grid,
        in_specs=[_part_spec(bs, H), _part_spec(bs, 16), _full_spec((1, H)),
                  _full_spec((H, H)), _full_spec((H, 3 * H)),
                  _full_spec((1, 3 * H))],
        out_specs=[_rows_spec(bs), _rows_spec(bs), _rows_spec(bs, 3 * H)],
        out_shape=[jax.ShapeDtypeStruct((npad, H), jnp.float32),
                   jax.ShapeDtypeStruct((npad, H), jnp.float32),
                   jax.ShapeDtypeStruct((npad, 3 * H), jnp.float32)],
    )(num, den, b, w0, whhT, bhh)


def _gru_step(m, x, gh, wihT, bih, npad, bs, last, w1=None, whhT=None, bhh=None):
    grid = (npad // bs,)

    def body(m_ref, x_ref, gh_ref, wihT_ref, bih_ref, *rest):
        mm = m_ref[0] + m_ref[1]
        gi = (jnp.dot(mm, wihT_ref[...], preferred_element_type=jnp.float32)
              + bih_ref[...])
        gh_v = gh_ref[...]
        r = jax.nn.sigmoid(gi[:, :H] + gh_v[:, :H])
        z = jax.nn.sigmoid(gi[:, H:2 * H] + gh_v[:, H:2 * H])
        n = jnp.tanh(gi[:, 2 * H:] + r * gh_v[:, 2 * H:])
        x2 = (1.0 - z) * n + z * x_ref[...]
        if last:
            (out_ref,) = rest
            out_ref[...] = jnp.maximum(x2, 0.0)
        else:
            w1_ref, whhT_ref, bhh_ref, x2_ref, y2_ref, gh2_ref = rest
            x2_ref[...] = x2
            y2_ref[...] = jnp.dot(x2, w1_ref[...], preferred_element_type=jnp.float32)
            gh2_ref[...] = (jnp.dot(x2, whhT_ref[...], preferred_element_type=jnp.float32)
                            + bhh_ref[...])

    in_specs = [_part_spec(bs, H), _rows_spec(bs), _rows_spec(bs, 3 * H),
                _full_spec((H, 3 * H)), _full_spec((1, 3 * H))]
    args = [m, x, gh, wihT, bih]
    if last:
        out_specs = [_rows_spec(bs)]
        out_shape = [jax.ShapeDtypeStruct((npad, H), jnp.float32)]
    else:
        in_specs += [_full_spec((H, H)), _full_spec((H, 3 * H)),
                     _full_spec((1, 3 * H))]
        args += [w1, whhT, bhh]
        out_specs = [_rows_spec(bs), _rows_spec(bs), _rows_spec(bs, 3 * H)]
        out_shape = [jax.ShapeDtypeStruct((npad, H), jnp.float32),
                     jax.ShapeDtypeStruct((npad, H), jnp.float32),
                     jax.ShapeDtypeStruct((npad, 3 * H), jnp.float32)]

    return pl.pallas_call(body, grid=grid, in_specs=in_specs,
                          out_specs=out_specs, out_shape=out_shape)(*args)


def _glob_combine(num, den, b, rwT, rb, npad, bs):
    grid = (npad // bs,)

    def body(num_ref, den_ref, b_ref, rwT_ref, rb_ref, xg_ref, rt_ref):
        nm = num_ref[0] + num_ref[1]
        dn = den_ref[0][:, 0:1] + den_ref[1][:, 0:1]
        xg = nm / (dn + 1e-16) + b_ref[...]
        xg_ref[...] = xg
        rt_ref[...] = (jnp.dot(xg, rwT_ref[...], preferred_element_type=jnp.float32)
                       + rb_ref[...])

    return pl.pallas_call(
        body,
        grid=grid,
        in_specs=[_part_spec(bs, H), _part_spec(bs, 16), _full_spec((1, H)),
                  _full_spec((H, H)), _full_spec((1, H))],
        out_specs=[_rows_spec(bs), _rows_spec(bs)],
        out_shape=[jax.ShapeDtypeStruct((npad, H), jnp.float32),
                   jax.ShapeDtypeStruct((npad, H), jnp.float32)],
    )(num, den, b, rwT, rb)


def _sage_final(s, cden, rt, lwT, lb, nout, bs):
    grid = (nout // bs,)

    def body(s_ref, c_ref, rt_ref, lwT_ref, lb_ref, out_ref):
        sm = s_ref[0] + s_ref[1]
        cnt = c_ref[0][:, 0:1] + c_ref[1][:, 0:1]
        mean = sm / jnp.maximum(cnt, 1.0)
        out = (jnp.dot(mean, lwT_ref[...], preferred_element_type=jnp.float32)
               + lb_ref[...] + rt_ref[...])
        nrm = jnp.sqrt(jnp.sum(out * out, axis=1, keepdims=True))
        out_ref[...] = out / jnp.maximum(nrm, 1e-12)

    return pl.pallas_call(
        body,
        grid=grid,
        in_specs=[_part_spec(bs, H), _part_spec(bs, 16), _rows_spec(bs),
                  _full_spec((H, H)), _full_spec((1, H))],
        out_specs=_rows_spec(bs),
        out_shape=jax.ShapeDtypeStruct((nout, H), jnp.float32),
    )(s, cden, rt, lwT, lb)


def _pad_edges(src, dst, epad, junk):
    n = src.shape[0]
    pad = epad - n
    src = jnp.concatenate([src.astype(jnp.int32), jnp.zeros((pad,), jnp.int32)])
    dst = jnp.concatenate([dst.astype(jnp.int32),
                           jnp.full((pad,), junk, jnp.int32)])
    return src, dst


def kernel(inputs, A, edge_index, node_idx, g_ei1, g_ei2, params):
    p = params
    emb = p['embedding']

    inputs = inputs.astype(jnp.int32)
    ei = edge_index.astype(jnp.int32)
    ge2 = g_ei2.astype(jnp.int32)
    nidx4 = node_idx[:NG].astype(jnp.int32)

    gidx = jnp.concatenate([
        inputs,
        jnp.zeros((NPAD_S - NSESS,), jnp.int32),
        nidx4,
        jnp.zeros((BGATHER - NPAD_S - NG,), jnp.int32),
    ])

    loop_s = jnp.arange(NSESS, dtype=jnp.int32)
    src_s, dst_s = _pad_edges(jnp.concatenate([ei[0], loop_s]),
                              jnp.concatenate([ei[1], loop_s]),
                              EPAD_GAT_S, NSESS)
    src_g1, dst_g1 = _pad_edges(ei[0], ei[1], EPAD_GGC, NSESS)

    loop_g = jnp.arange(NG, dtype=jnp.int32)
    src_s2, dst_s2 = _pad_edges(jnp.concatenate([ge2[0], loop_g]),
                                jnp.concatenate([ge2[1], loop_g]),
                                EPAD_GAT_G, NG)
    src_sg, dst_sg = _pad_edges(ge2[0], ge2[1], EPAD_SAGE, NG)

    rows = _make_gather(BGATHER)(emb, gidx)
    h0 = rows[:NPAD_S]
    x4 = rows[NPAD_S:NPAD_S + NPAD_G]
    pad_row = rows[NSESS:NSESS + 1]

    xw1, as1, ad1 = _gat_prep(h0, p['g1_W'],
                              p['g1_asrc'][None, :], p['g1_adst'][None, :],
                              NPAD_S, 512)
    num1, den1 = _make_agg(NPAD_S, EPAD_GAT_S, NPAD_S, True)(
        xw1, as1, ad1, src_s, dst_s)
    h1, y1, gh1 = _sess_combine(num1, den1, p['g1_b'][None, :],
                                p['gg_weight'][0], p['gg_whh'].T,
                                p['gg_bhh'][None, :], NPAD_S, 512)
    m1, _ = _make_agg(NPAD_S, EPAD_GGC, NPAD_S, False)(
        y1, src_g1, dst_g1)
    x2, y2, gh2 = _gru_step(m1, h1, gh1, p['gg_wih'].T, p['gg_bih'][None, :],
                            NPAD_S, 512, last=False,
                            w1=p['gg_weight'][1], whhT=p['gg_whh'].T,
                            bhh=p['gg_bhh'][None, :])
    m2, _ = _make_agg(NPAD_S, EPAD_GGC, NPAD_S, False)(
        y2, src_g1, dst_g1)
    hidden_full = _gru_step(m2, x2, gh2, p['gg_wih'].T, p['gg_bih'][None, :],
                            NPAD_S, 512, last=True)[0]
    hidden = hidden_full[:NSESS]

    xwg, asg, adg = _gat_prep(x4, p['gat2_W'],
                              p['gat2_asrc'][None, :], p['gat2_adst'][None, :],
                              NPAD_G, 512)
    numg, deng = _make_agg(NPAD_G, EPAD_GAT_G, NPAD_G, True)(
        xwg, asg, adg, src_s2, dst_s2)
    xg, rt = _glob_combine(numg, deng, p['gat2_b'][None, :],
                           p['s2_rw'].T, p['s2_rb'][None, :], NPAD_G, 512)
    sg, cg = _make_agg(NPAD_G, EPAD_SAGE, NPAD_G, False)(
        xg, src_sg, dst_sg)
    g_h = _sage_final(sg, cg, rt, p['s2_lw'].T, p['s2_lb'][None, :], NG, 512)

    return (hidden, pad_row, g_h)
